# baseline (speedup 1.0000x reference)
import functools

import jax
import jax.numpy as jnp
from jax.experimental import pallas as pl
from jax.experimental.pallas import tpu as pltpu

N_NODES = 50000
ROW_BLOCK = 2000


def _rsqrt_clip(x):
    return jax.lax.rsqrt(jnp.clip(x, 1.0, None))


def _tc1_body(inp_ref, hid_ref, odeg_ref, w1_ref, h1a_ref, h1b_ref):
    onorm = _rsqrt_clip(odeg_ref[...])
    xi = inp_ref[...] * onorm
    xh = hid_ref[...] * onorm
    w1 = w1_ref[...]
    h1a_ref[...] = (jnp.dot(xi, w1[:64, :32], preferred_element_type=jnp.float32)
                    + jnp.dot(xh, w1[64:, :32], preferred_element_type=jnp.float32))
    h1b_ref[...] = (jnp.dot(xi, w1[:64, 32:], preferred_element_type=jnp.float32)
                    + jnp.dot(xh, w1[64:, 32:], preferred_element_type=jnp.float32))


def _tc1(inp, hid, out_deg, W1):
    n = inp.shape[0]
    grid = n // ROW_BLOCK
    return pl.pallas_call(
        _tc1_body,
        grid=(grid,),
        in_specs=[
            pl.BlockSpec((ROW_BLOCK, 64), lambda i: (i, 0)),
            pl.BlockSpec((ROW_BLOCK, 32), lambda i: (i, 0)),
            pl.BlockSpec((ROW_BLOCK, 1), lambda i: (i, 0)),
            pl.BlockSpec((96, 64), lambda i: (0, 0)),
        ],
        out_specs=[
            pl.BlockSpec((ROW_BLOCK, 32), lambda i: (i, 0)),
            pl.BlockSpec((ROW_BLOCK, 32), lambda i: (i, 0)),
        ],
        out_shape=[
            jax.ShapeDtypeStruct((n, 32), jnp.float32),
            jax.ShapeDtypeStruct((n, 32), jnp.float32),
        ],
    )(inp, hid, out_deg, W1)


def _tc2_body(inp_ref, hid_ref, agg1a_ref, agg1b_ref, ideg_ref, odeg_ref,
              w2_ref, b1_ref, h2_ref, upd_ref):
    innorm = _rsqrt_clip(ideg_ref[...])
    onorm = _rsqrt_clip(odeg_ref[...])
    b1 = b1_ref[...]
    reset = jax.nn.sigmoid(agg1a_ref[...] * innorm + b1[:, :32])
    upd = jax.nn.sigmoid(agg1b_ref[...] * innorm + b1[:, 32:])
    upd_ref[...] = upd
    xi = inp_ref[...] * onorm
    xh = reset * hid_ref[...] * onorm
    w2 = w2_ref[...]
    h2_ref[...] = (jnp.dot(xi, w2[:64, :], preferred_element_type=jnp.float32)
                   + jnp.dot(xh, w2[64:, :], preferred_element_type=jnp.float32))


def _tc2(inp, hid, agg1a, agg1b, in_deg, out_deg, W2, b1):
    n = inp.shape[0]
    grid = n // ROW_BLOCK
    return pl.pallas_call(
        _tc2_body,
        grid=(grid,),
        in_specs=[
            pl.BlockSpec((ROW_BLOCK, 64), lambda i: (i, 0)),
            pl.BlockSpec((ROW_BLOCK, 32), lambda i: (i, 0)),
            pl.BlockSpec((ROW_BLOCK, 32), lambda i: (i, 0)),
            pl.BlockSpec((ROW_BLOCK, 32), lambda i: (i, 0)),
            pl.BlockSpec((ROW_BLOCK, 1), lambda i: (i, 0)),
            pl.BlockSpec((ROW_BLOCK, 1), lambda i: (i, 0)),
            pl.BlockSpec((96, 32), lambda i: (0, 0)),
            pl.BlockSpec((1, 64), lambda i: (0, 0)),
        ],
        out_specs=[
            pl.BlockSpec((ROW_BLOCK, 32), lambda i: (i, 0)),
            pl.BlockSpec((ROW_BLOCK, 32), lambda i: (i, 0)),
        ],
        out_shape=[
            jax.ShapeDtypeStruct((n, 32), jnp.float32),
            jax.ShapeDtypeStruct((n, 32), jnp.float32),
        ],
    )(inp, hid, agg1a, agg1b, in_deg, out_deg, W2, b1)


def _tc3_body(hid_ref, upd_ref, agg2a_ref, agg2b_ref, ideg_ref, b2_ref, out_ref):
    innorm = _rsqrt_clip(ideg_ref[...])
    cand = jnp.tanh((agg2a_ref[...] + agg2b_ref[...]) * innorm + b2_ref[...])
    upd = upd_ref[...]
    out_ref[...] = upd * hid_ref[...] + (1.0 - upd) * cand


def _tc3(hid, upd, agg2a, agg2b, in_deg, b2):
    n = hid.shape[0]
    grid = n // ROW_BLOCK
    return pl.pallas_call(
        _tc3_body,
        grid=(grid,),
        in_specs=[
            pl.BlockSpec((ROW_BLOCK, 32), lambda i: (i, 0)),
            pl.BlockSpec((ROW_BLOCK, 32), lambda i: (i, 0)),
            pl.BlockSpec((ROW_BLOCK, 32), lambda i: (i, 0)),
            pl.BlockSpec((ROW_BLOCK, 32), lambda i: (i, 0)),
            pl.BlockSpec((ROW_BLOCK, 1), lambda i: (i, 0)),
            pl.BlockSpec((1, 32), lambda i: (0, 0)),
        ],
        out_specs=pl.BlockSpec((ROW_BLOCK, 32), lambda i: (i, 0)),
        out_shape=jax.ShapeDtypeStruct((n, 32), jnp.float32),
    )(hid, upd, agg2a, agg2b, in_deg, b2)


def kernel(inputs, hidden_state, edge_index, W1, b1, W2, b2):
    n = inputs.shape[0]
    src = edge_index[0].astype(jnp.int32)
    dst = edge_index[1].astype(jnp.int32)

    out_deg = jnp.zeros((n,), jnp.float32).at[src].add(1.0).reshape(n, 1)
    in_deg = jnp.zeros((n,), jnp.float32).at[dst].add(1.0).reshape(n, 1)

    h1a, h1b = _tc1(inputs, hidden_state, out_deg, W1)

    agg1a = jnp.zeros((n, 32), jnp.float32).at[dst].add(h1a[src])
    agg1b = jnp.zeros((n, 32), jnp.float32).at[dst].add(h1b[src])

    h2, upd = _tc2(inputs, hidden_state, agg1a, agg1b, in_deg, out_deg, W2,
                   b1.reshape(1, 64))

    agg2a = jnp.zeros((n, 32), jnp.float32).at[dst].add(h2[src])
    agg2b = jnp.zeros((n, 32), jnp.float32)

    new_h = _tc3(hidden_state, upd, agg2a, agg2b, in_deg, b2.reshape(1, 32))
    return (new_h, new_h)


# speedup vs baseline: 3.7269x; 3.7269x over previous
import functools

import jax
import jax.numpy as jnp
from jax import lax
from jax.experimental import pallas as pl
from jax.experimental.pallas import tpu as pltpu
from jax.experimental.pallas import tpu_sc as plsc

N = 50000
E = 800000
CHUNK = 128
NS = 16
NC = 2
E_PAD = ((E + NC * NS * CHUNK - 1) // (NC * NS * CHUNK)) * (NC * NS * CHUNK)
EROWS = E_PAD // CHUNK
ROWS_PER_TILE_FULL = EROWS // NS
ROWS_PER_TILE_HALF = EROWS // (NC * NS)
N_PAD = N + CHUNK - (N % CHUNK) if N % CHUNK else N
NP_TILE = N_PAD // NS
ROW_BLOCK = 2000

_mesh = plsc.VectorSubcoreMesh(core_axis_name="c", subcore_axis_name="s")


def _rsqrt_clip(x):
    return jax.lax.rsqrt(jnp.clip(x, 1.0, None))


@functools.partial(
    pl.kernel,
    out_type=jax.ShapeDtypeStruct((NC, N_PAD), jnp.float32),
    mesh=_mesh,
    compiler_params=pltpu.CompilerParams(use_tc_tiling_on_sc=False),
    scratch_types=[
        pltpu.VMEM((ROWS_PER_TILE_FULL, CHUNK), jnp.int32),
        pltpu.VMEM((CHUNK,), jnp.float32),
        pltpu.VMEM_SHARED((N_PAD,), jnp.float32),
    ],
)
def _sc_degrees(edges3d, zr1, degs, idx, ones_v, acc):
    c = lax.axis_index("c")
    s = lax.axis_index("s")
    pltpu.sync_copy(edges3d.at[c, pl.ds(s * ROWS_PER_TILE_FULL, ROWS_PER_TILE_FULL)], idx)
    for i in range(CHUNK // 16):
        ones_v[pl.ds(16 * i, 16)] = jnp.full((16,), 1.0, jnp.float32)
    pltpu.sync_copy(zr1, acc.at[pl.ds(s * NP_TILE, NP_TILE)])
    plsc.subcore_barrier()

    def body(k, _):
        pltpu.sync_copy(ones_v, acc.at[idx.at[k]], add=True)
        return _

    lax.fori_loop(0, ROWS_PER_TILE_FULL, body, None)
    plsc.subcore_barrier()
    pltpu.sync_copy(acc.at[pl.ds(s * NP_TILE, NP_TILE)],
                    degs.at[c, pl.ds(s * NP_TILE, NP_TILE)])


def _make_agg(num_phases):
    @functools.partial(
        pl.kernel,
        out_type=[jax.ShapeDtypeStruct((NC, N_PAD, 8), jnp.float32)
                  for _ in range(num_phases)],
        mesh=_mesh,
        compiler_params=pltpu.CompilerParams(use_tc_tiling_on_sc=False),
        scratch_types=[
            pltpu.VMEM((ROWS_PER_TILE_FULL, CHUNK), jnp.int32),
            pltpu.VMEM((ROWS_PER_TILE_FULL, CHUNK), jnp.int32),
            pltpu.VMEM((2, CHUNK, 8), jnp.float32),
            pltpu.VMEM_SHARED((N_PAD, 8), jnp.float32),
            pltpu.SemaphoreType.DMA,
            pltpu.SemaphoreType.DMA,
        ],
    )
    def agg_kernel(*refs):
        hs = refs[:num_phases]
        edges3d, zr8 = refs[num_phases], refs[num_phases + 1]
        outs = refs[num_phases + 2:2 * num_phases + 2]
        sidx, didx, rows, acc, sem0, sem1 = refs[2 * num_phases + 2:]
        c = lax.axis_index("c")
        s = lax.axis_index("s")
        ebase = s * ROWS_PER_TILE_FULL
        pltpu.sync_copy(edges3d.at[2, pl.ds(ebase, ROWS_PER_TILE_FULL)], sidx)
        pltpu.sync_copy(edges3d.at[1, pl.ds(ebase, ROWS_PER_TILE_FULL)], didx)
        sems = (sem0, sem1)

        for p in range(num_phases):
            hsrc = hs[p].at[c]
            pltpu.sync_copy(zr8, acc.at[pl.ds(s * NP_TILE, NP_TILE)])
            plsc.subcore_barrier()

            pltpu.async_copy(hsrc.at[sidx.at[0]], rows.at[0], sem0)

            def body(j, _):
                for b in range(2):
                    k = 2 * j + b
                    pltpu.make_async_copy(hsrc.at[sidx.at[k]], rows.at[b], sems[b]).wait()
                    if b == 0:
                        pltpu.async_copy(hsrc.at[sidx.at[k + 1]], rows.at[1], sem1)
                    else:
                        @pl.when(j < ROWS_PER_TILE_FULL // 2 - 1)
                        def _start():
                            pltpu.async_copy(hsrc.at[sidx.at[k + 1]], rows.at[0], sem0)
                    pltpu.sync_copy(rows.at[b], acc.at[didx.at[k]], add=True)
                return _

            lax.fori_loop(0, ROWS_PER_TILE_FULL // 2, body, None)
            plsc.subcore_barrier()
            pltpu.sync_copy(acc.at[pl.ds(s * NP_TILE, NP_TILE)],
                            outs[p].at[c, pl.ds(s * NP_TILE, NP_TILE)])
            plsc.subcore_barrier()

    return agg_kernel


_sc_agg4 = _make_agg(4)
_sc_agg2p = _make_agg(2)


def _tc1_body(inp_ref, hid_ref, odeg_ref, w1_ref, *h1_refs):
    onorm = _rsqrt_clip(odeg_ref[0])
    xi = inp_ref[...] * onorm
    xh = hid_ref[...] * onorm
    w1 = w1_ref[...]
    h1 = (jnp.dot(xi, w1[:64, :], preferred_element_type=jnp.float32)
          + jnp.dot(xh, w1[64:, :], preferred_element_type=jnp.float32))
    for p in range(4):
        h1_refs[p][...] = jnp.stack([h1[:, 16 * p:16 * p + 8],
                                     h1[:, 16 * p + 8:16 * p + 16]])


def _tc1(inp, hid, degs, W1):
    grid = N // ROW_BLOCK
    return pl.pallas_call(
        _tc1_body,
        grid=(grid,),
        in_specs=[
            pl.BlockSpec((ROW_BLOCK, 64), lambda i: (i, 0)),
            pl.BlockSpec((ROW_BLOCK, 32), lambda i: (i, 0)),
            pl.BlockSpec((1, ROW_BLOCK, 1), lambda i: (0, i, 0)),
            pl.BlockSpec((96, 64), lambda i: (0, 0)),
        ],
        out_specs=[pl.BlockSpec((2, ROW_BLOCK, 8), lambda i: (0, i, 0))
                   for _ in range(4)],
        out_shape=[jax.ShapeDtypeStruct((2, N, 8), jnp.float32)
                   for _ in range(4)],
    )(inp, hid, degs, W1)


def _tc2_body(inp_ref, hid_ref, a0_ref, a1_ref, a2_ref, a3_ref, degs_ref,
              w2_ref, b1_ref, h2a_ref, h2b_ref, upd_ref):
    onorm = _rsqrt_clip(degs_ref[0])
    innorm = _rsqrt_clip(degs_ref[1])
    b1 = b1_ref[...]
    agg1a = jnp.concatenate([a0_ref[0], a0_ref[1], a1_ref[0], a1_ref[1]], axis=-1)
    agg1b = jnp.concatenate([a2_ref[0], a2_ref[1], a3_ref[0], a3_ref[1]], axis=-1)
    reset = jax.nn.sigmoid(agg1a * innorm + b1[:, :32])
    upd = jax.nn.sigmoid(agg1b * innorm + b1[:, 32:])
    upd_ref[...] = upd
    xi = inp_ref[...] * onorm
    xh = reset * hid_ref[...] * onorm
    w2 = w2_ref[...]
    h2 = (jnp.dot(xi, w2[:64, :], preferred_element_type=jnp.float32)
          + jnp.dot(xh, w2[64:, :], preferred_element_type=jnp.float32))
    h2a_ref[...] = jnp.stack([h2[:, 0:8], h2[:, 8:16]])
    h2b_ref[...] = jnp.stack([h2[:, 16:24], h2[:, 24:32]])


def _tc2(inp, hid, a0, a1, a2, a3, degs, W2, b1):
    grid = N // ROW_BLOCK
    return pl.pallas_call(
        _tc2_body,
        grid=(grid,),
        in_specs=[
            pl.BlockSpec((ROW_BLOCK, 64), lambda i: (i, 0)),
            pl.BlockSpec((ROW_BLOCK, 32), lambda i: (i, 0)),
            pl.BlockSpec((2, ROW_BLOCK, 8), lambda i: (0, i, 0)),
            pl.BlockSpec((2, ROW_BLOCK, 8), lambda i: (0, i, 0)),
            pl.BlockSpec((2, ROW_BLOCK, 8), lambda i: (0, i, 0)),
            pl.BlockSpec((2, ROW_BLOCK, 8), lambda i: (0, i, 0)),
            pl.BlockSpec((2, ROW_BLOCK, 1), lambda i: (0, i, 0)),
            pl.BlockSpec((96, 32), lambda i: (0, 0)),
            pl.BlockSpec((1, 64), lambda i: (0, 0)),
        ],
        out_specs=[
            pl.BlockSpec((2, ROW_BLOCK, 8), lambda i: (0, i, 0)),
            pl.BlockSpec((2, ROW_BLOCK, 8), lambda i: (0, i, 0)),
            pl.BlockSpec((ROW_BLOCK, 32), lambda i: (i, 0)),
        ],
        out_shape=[
            jax.ShapeDtypeStruct((2, N, 8), jnp.float32),
            jax.ShapeDtypeStruct((2, N, 8), jnp.float32),
            jax.ShapeDtypeStruct((N, 32), jnp.float32),
        ],
    )(inp, hid, a0, a1, a2, a3, degs, W2, b1)


def _tc3_body(hid_ref, upd_ref, g0_ref, g1_ref, degs_ref, b2_ref, out_ref):
    innorm = _rsqrt_clip(degs_ref[0])
    agg2 = jnp.concatenate([g0_ref[0], g0_ref[1], g1_ref[0], g1_ref[1]], axis=-1)
    cand = jnp.tanh(agg2 * innorm + b2_ref[...])
    upd = upd_ref[...]
    out_ref[...] = upd * hid_ref[...] + (1.0 - upd) * cand


def _tc3(hid, upd, g0, g1, degs, b2):
    grid = N // ROW_BLOCK
    return pl.pallas_call(
        _tc3_body,
        grid=(grid,),
        in_specs=[
            pl.BlockSpec((ROW_BLOCK, 32), lambda i: (i, 0)),
            pl.BlockSpec((ROW_BLOCK, 32), lambda i: (i, 0)),
            pl.BlockSpec((2, ROW_BLOCK, 8), lambda i: (0, i, 0)),
            pl.BlockSpec((2, ROW_BLOCK, 8), lambda i: (0, i, 0)),
            pl.BlockSpec((1, ROW_BLOCK, 1), lambda i: (1, i, 0)),
            pl.BlockSpec((1, 32), lambda i: (0, 0)),
        ],
        out_specs=pl.BlockSpec((ROW_BLOCK, 32), lambda i: (i, 0)),
        out_shape=jax.ShapeDtypeStruct((N, 32), jnp.float32),
    )(hid, upd, g0, g1, degs, b2)


def kernel(inputs, hidden_state, edge_index, W1, b1, W2, b2):
    src = edge_index[0].astype(jnp.int32)
    dst = edge_index[1].astype(jnp.int32)

    pad = E_PAD - E
    pad_deg = jnp.full((pad,), N, jnp.int32)
    pad_gat = jnp.full((pad,), N - 1, jnp.int32)
    edges3d = jnp.stack([
        jnp.concatenate([src, pad_deg]),
        jnp.concatenate([dst, pad_deg]),
        jnp.concatenate([src, pad_gat]),
    ]).reshape(3, EROWS, CHUNK)

    zr1 = jnp.zeros((NP_TILE,), jnp.float32)
    zr8 = jnp.zeros((NP_TILE, 8), jnp.float32)

    degs = _sc_degrees(edges3d, zr1).reshape(NC, N_PAD, 1)

    h1p = _tc1(inputs, hidden_state, degs, W1)
    a0, a1, a2, a3 = _sc_agg4(*h1p, edges3d, zr8)

    h2a, h2b, upd = _tc2(inputs, hidden_state, a0, a1, a2, a3, degs, W2,
                         b1.reshape(1, 64))
    g0, g1 = _sc_agg2p(h2a, h2b, edges3d, zr8)

    new_h = _tc3(hidden_state, upd, g0, g1, degs, b2.reshape(1, 32))
    return (new_h, new_h)


# speedup vs baseline: 6.2826x; 1.6857x over previous
import functools

import jax
import jax.numpy as jnp
from jax import lax
from jax.experimental import pallas as pl
from jax.experimental.pallas import tpu as pltpu
from jax.experimental.pallas import tpu_sc as plsc

N = 50000
E = 800000
CHUNK = 128
NS = 16
NC = 2
E_PAD = ((E + NC * NS * CHUNK - 1) // (NC * NS * CHUNK)) * (NC * NS * CHUNK)
EROWS = E_PAD // CHUNK
ROWS_PER_TILE_FULL = EROWS // NS
ROWS_PER_TILE_HALF = EROWS // (NC * NS)
N_PAD = N + CHUNK - (N % CHUNK) if N % CHUNK else N
NP_TILE = N_PAD // NS
ROW_BLOCK = 2000

_mesh = plsc.VectorSubcoreMesh(core_axis_name="c", subcore_axis_name="s")


def _rsqrt_clip(x):
    return jax.lax.rsqrt(jnp.clip(x, 1.0, None))


@functools.partial(
    pl.kernel,
    out_type=jax.ShapeDtypeStruct((NC, N_PAD), jnp.float32),
    mesh=_mesh,
    compiler_params=pltpu.CompilerParams(use_tc_tiling_on_sc=False),
    scratch_types=[
        pltpu.VMEM((ROWS_PER_TILE_FULL, CHUNK), jnp.int32),
        pltpu.VMEM((CHUNK,), jnp.float32),
        pltpu.VMEM_SHARED((N_PAD,), jnp.float32),
    ],
)
def _sc_degrees(edges3d, zr1, degs, idx, ones_v, acc):
    c = lax.axis_index("c")
    s = lax.axis_index("s")
    pltpu.sync_copy(edges3d.at[c, pl.ds(s * ROWS_PER_TILE_FULL, ROWS_PER_TILE_FULL)], idx)
    for i in range(CHUNK // 16):
        ones_v[pl.ds(16 * i, 16)] = jnp.full((16,), 1.0, jnp.float32)
    pltpu.sync_copy(zr1, acc.at[pl.ds(s * NP_TILE, NP_TILE)])
    plsc.subcore_barrier()

    def body(k, _):
        pltpu.sync_copy(ones_v, acc.at[idx.at[k]], add=True)
        return _

    lax.fori_loop(0, ROWS_PER_TILE_FULL, body, None)
    plsc.subcore_barrier()
    pltpu.sync_copy(acc.at[pl.ds(s * NP_TILE, NP_TILE)],
                    degs.at[c, pl.ds(s * NP_TILE, NP_TILE)])


def _make_agg(num_phases):
    @functools.partial(
        pl.kernel,
        out_type=[jax.ShapeDtypeStruct((NC, N_PAD, 8), jnp.float32)
                  for _ in range(num_phases)],
        mesh=_mesh,
        compiler_params=pltpu.CompilerParams(use_tc_tiling_on_sc=False),
        scratch_types=[
            pltpu.VMEM((ROWS_PER_TILE_FULL, CHUNK), jnp.int32),
            pltpu.VMEM((ROWS_PER_TILE_FULL, CHUNK), jnp.int32),
            pltpu.VMEM((4, CHUNK, 8), jnp.float32),
            pltpu.VMEM_SHARED((N_PAD, 8), jnp.float32),
        ] + [pltpu.SemaphoreType.DMA] * 8,
    )
    def agg_kernel(*refs):
        hs = refs[:num_phases]
        edges3d, zr8 = refs[num_phases], refs[num_phases + 1]
        outs = refs[num_phases + 2:2 * num_phases + 2]
        rest = refs[2 * num_phases + 2:]
        sidx, didx, rows, acc = rest[:4]
        sg = rest[4:8]
        ss = rest[8:12]
        c = lax.axis_index("c")
        s = lax.axis_index("s")
        ebase = s * ROWS_PER_TILE_FULL
        pltpu.sync_copy(edges3d.at[2, pl.ds(ebase, ROWS_PER_TILE_FULL)], sidx)
        pltpu.sync_copy(edges3d.at[1, pl.ds(ebase, ROWS_PER_TILE_FULL)], didx)
        TOT = ROWS_PER_TILE_FULL

        for p in range(num_phases):
            hsrc = hs[p].at[c]
            pltpu.sync_copy(zr8, acc.at[pl.ds(s * NP_TILE, NP_TILE)])
            plsc.subcore_barrier()

            pltpu.async_copy(hsrc.at[sidx.at[0]], rows.at[0], sg[0])
            pltpu.async_copy(hsrc.at[sidx.at[1]], rows.at[1], sg[1])

            def body(q, _):
                for j in range(4):
                    k = 4 * q + j
                    b = j
                    bb = (j + 2) % 4
                    @pl.when(k >= 2)
                    def _wait_scat():
                        pltpu.make_async_copy(
                            rows.at[bb], acc.at[didx.at[k - 2]], ss[bb]).wait()
                    @pl.when(k + 2 < TOT)
                    def _start_gather():
                        pltpu.async_copy(hsrc.at[sidx.at[k + 2]], rows.at[bb], sg[bb])
                    pltpu.make_async_copy(hsrc.at[sidx.at[k]], rows.at[b], sg[b]).wait()
                    pltpu.async_copy(rows.at[b], acc.at[didx.at[k]], ss[b], add=True)
                return _

            lax.fori_loop(0, TOT // 4, body, None)
            pltpu.make_async_copy(rows.at[2], acc.at[didx.at[TOT - 2]], ss[2]).wait()
            pltpu.make_async_copy(rows.at[3], acc.at[didx.at[TOT - 1]], ss[3]).wait()
            plsc.subcore_barrier()
            pltpu.sync_copy(acc.at[pl.ds(s * NP_TILE, NP_TILE)],
                            outs[p].at[c, pl.ds(s * NP_TILE, NP_TILE)])
            plsc.subcore_barrier()

    return agg_kernel


_sc_agg4 = _make_agg(4)
_sc_agg2p = _make_agg(2)


def _tc1_body(inp_ref, hid_ref, odeg_ref, w1_ref, *h1_refs):
    onorm = _rsqrt_clip(odeg_ref[0])
    xi = inp_ref[...] * onorm
    xh = hid_ref[...] * onorm
    w1 = w1_ref[...]
    h1 = (jnp.dot(xi, w1[:64, :], preferred_element_type=jnp.float32)
          + jnp.dot(xh, w1[64:, :], preferred_element_type=jnp.float32))
    for p in range(4):
        h1_refs[p][...] = jnp.stack([h1[:, 16 * p:16 * p + 8],
                                     h1[:, 16 * p + 8:16 * p + 16]])


def _tc1(inp, hid, degs, W1):
    grid = N // ROW_BLOCK
    return pl.pallas_call(
        _tc1_body,
        grid=(grid,),
        in_specs=[
            pl.BlockSpec((ROW_BLOCK, 64), lambda i: (i, 0)),
            pl.BlockSpec((ROW_BLOCK, 32), lambda i: (i, 0)),
            pl.BlockSpec((1, ROW_BLOCK, 1), lambda i: (0, i, 0)),
            pl.BlockSpec((96, 64), lambda i: (0, 0)),
        ],
        out_specs=[pl.BlockSpec((2, ROW_BLOCK, 8), lambda i: (0, i, 0))
                   for _ in range(4)],
        out_shape=[jax.ShapeDtypeStruct((2, N, 8), jnp.float32)
                   for _ in range(4)],
    )(inp, hid, degs, W1)


def _tc2_body(inp_ref, hid_ref, a0_ref, a1_ref, a2_ref, a3_ref, degs_ref,
              w2_ref, b1_ref, h2a_ref, h2b_ref, upd_ref):
    onorm = _rsqrt_clip(degs_ref[0])
    innorm = _rsqrt_clip(degs_ref[1])
    b1 = b1_ref[...]
    agg1a = jnp.concatenate([a0_ref[0], a0_ref[1], a1_ref[0], a1_ref[1]], axis=-1)
    agg1b = jnp.concatenate([a2_ref[0], a2_ref[1], a3_ref[0], a3_ref[1]], axis=-1)
    reset = jax.nn.sigmoid(agg1a * innorm + b1[:, :32])
    upd = jax.nn.sigmoid(agg1b * innorm + b1[:, 32:])
    upd_ref[...] = upd
    xi = inp_ref[...] * onorm
    xh = reset * hid_ref[...] * onorm
    w2 = w2_ref[...]
    h2 = (jnp.dot(xi, w2[:64, :], preferred_element_type=jnp.float32)
          + jnp.dot(xh, w2[64:, :], preferred_element_type=jnp.float32))
    h2a_ref[...] = jnp.stack([h2[:, 0:8], h2[:, 8:16]])
    h2b_ref[...] = jnp.stack([h2[:, 16:24], h2[:, 24:32]])


def _tc2(inp, hid, a0, a1, a2, a3, degs, W2, b1):
    grid = N // ROW_BLOCK
    return pl.pallas_call(
        _tc2_body,
        grid=(grid,),
        in_specs=[
            pl.BlockSpec((ROW_BLOCK, 64), lambda i: (i, 0)),
            pl.BlockSpec((ROW_BLOCK, 32), lambda i: (i, 0)),
            pl.BlockSpec((2, ROW_BLOCK, 8), lambda i: (0, i, 0)),
            pl.BlockSpec((2, ROW_BLOCK, 8), lambda i: (0, i, 0)),
            pl.BlockSpec((2, ROW_BLOCK, 8), lambda i: (0, i, 0)),
            pl.BlockSpec((2, ROW_BLOCK, 8), lambda i: (0, i, 0)),
            pl.BlockSpec((2, ROW_BLOCK, 1), lambda i: (0, i, 0)),
            pl.BlockSpec((96, 32), lambda i: (0, 0)),
            pl.BlockSpec((1, 64), lambda i: (0, 0)),
        ],
        out_specs=[
            pl.BlockSpec((2, ROW_BLOCK, 8), lambda i: (0, i, 0)),
            pl.BlockSpec((2, ROW_BLOCK, 8), lambda i: (0, i, 0)),
            pl.BlockSpec((ROW_BLOCK, 32), lambda i: (i, 0)),
        ],
        out_shape=[
            jax.ShapeDtypeStruct((2, N, 8), jnp.float32),
            jax.ShapeDtypeStruct((2, N, 8), jnp.float32),
            jax.ShapeDtypeStruct((N, 32), jnp.float32),
        ],
    )(inp, hid, a0, a1, a2, a3, degs, W2, b1)


def _tc3_body(hid_ref, upd_ref, g0_ref, g1_ref, degs_ref, b2_ref, out_ref):
    innorm = _rsqrt_clip(degs_ref[0])
    agg2 = jnp.concatenate([g0_ref[0], g0_ref[1], g1_ref[0], g1_ref[1]], axis=-1)
    cand = jnp.tanh(agg2 * innorm + b2_ref[...])
    upd = upd_ref[...]
    out_ref[...] = upd * hid_ref[...] + (1.0 - upd) * cand


def _tc3(hid, upd, g0, g1, degs, b2):
    grid = N // ROW_BLOCK
    return pl.pallas_call(
        _tc3_body,
        grid=(grid,),
        in_specs=[
            pl.BlockSpec((ROW_BLOCK, 32), lambda i: (i, 0)),
            pl.BlockSpec((ROW_BLOCK, 32), lambda i: (i, 0)),
            pl.BlockSpec((2, ROW_BLOCK, 8), lambda i: (0, i, 0)),
            pl.BlockSpec((2, ROW_BLOCK, 8), lambda i: (0, i, 0)),
            pl.BlockSpec((1, ROW_BLOCK, 1), lambda i: (1, i, 0)),
            pl.BlockSpec((1, 32), lambda i: (0, 0)),
        ],
        out_specs=pl.BlockSpec((ROW_BLOCK, 32), lambda i: (i, 0)),
        out_shape=jax.ShapeDtypeStruct((N, 32), jnp.float32),
    )(hid, upd, g0, g1, degs, b2)


def kernel(inputs, hidden_state, edge_index, W1, b1, W2, b2):
    src = edge_index[0].astype(jnp.int32)
    dst = edge_index[1].astype(jnp.int32)

    pad = E_PAD - E
    pad_deg = jnp.full((pad,), N, jnp.int32)
    pad_gat = jnp.full((pad,), N - 1, jnp.int32)
    edges3d = jnp.stack([
        jnp.concatenate([src, pad_deg]),
        jnp.concatenate([dst, pad_deg]),
        jnp.concatenate([src, pad_gat]),
    ]).reshape(3, EROWS, CHUNK)

    zr1 = jnp.zeros((NP_TILE,), jnp.float32)
    zr8 = jnp.zeros((NP_TILE, 8), jnp.float32)

    degs = _sc_degrees(edges3d, zr1).reshape(NC, N_PAD, 1)

    h1p = _tc1(inputs, hidden_state, degs, W1)
    a0, a1, a2, a3 = _sc_agg4(*h1p, edges3d, zr8)

    h2a, h2b, upd = _tc2(inputs, hidden_state, a0, a1, a2, a3, degs, W2,
                         b1.reshape(1, 64))
    g0, g1 = _sc_agg2p(h2a, h2b, edges3d, zr8)

    new_h = _tc3(hidden_state, upd, g0, g1, degs, b2.reshape(1, 32))
    return (new_h, new_h)
